# baseline (speedup 1.0000x reference)
import functools

import jax
import jax.numpy as jnp
from jax import lax
from jax.experimental import pallas as pl
from jax.experimental.pallas import tpu as pltpu
from jax.experimental.pallas import tpu_sc as plsc

VOCAB = 100000
EMBED = 128
BATCH = 1024
SEQ = 512
TYPE_VOCAB = 2

L = 16
NW = 32
N = BATCH * SEQ
ROWS_PER_W = N // NW
CHUNK = 128
NCHUNK = ROWS_PER_W // CHUNK
NSLOT = 2
POS_PERIOD = SEQ // CHUNK
GVECS = CHUNK // L


def _body(ids_hbm, tt_hbm, word_hbm, pos_hbm, ttab_hbm, out_hbm,
          idx_v, tti_v, ramp_v, wbuf_v, pbuf_v, tbuf_v, gsems, ssems):
    wid = lax.axis_index("s") * 2 + lax.axis_index("c")
    wbase = wid * ROWS_PER_W

    for j in range(SEQ // L):
        ramp_v[pl.ds(j * L, L)] = lax.iota(jnp.int32, L) + (j * L)

    def issue(c, slot):
        base = wbase + c * CHUNK
        pos_off = (c % POS_PERIOD) * CHUNK
        pltpu.sync_copy(ids_hbm.at[pl.ds(base, CHUNK)], idx_v.at[slot])
        pltpu.sync_copy(tt_hbm.at[pl.ds(base, CHUNK)], tti_v.at[slot])
        pltpu.async_copy(word_hbm.at[idx_v.at[slot]], wbuf_v.at[slot],
                         gsems.at[slot])
        pltpu.async_copy(pos_hbm.at[ramp_v.at[pl.ds(pos_off, CHUNK)]],
                         pbuf_v.at[slot], gsems.at[slot])
        pltpu.async_copy(ttab_hbm.at[tti_v.at[slot]], tbuf_v.at[slot],
                         gsems.at[slot])

    def wait_gathers(c, slot):
        base = wbase + c * CHUNK
        pos_off = (c % POS_PERIOD) * CHUNK
        pltpu.make_async_copy(word_hbm.at[idx_v.at[slot]], wbuf_v.at[slot],
                              gsems.at[slot]).wait()
        pltpu.make_async_copy(pos_hbm.at[ramp_v.at[pl.ds(pos_off, CHUNK)]],
                              pbuf_v.at[slot], gsems.at[slot]).wait()
        pltpu.make_async_copy(ttab_hbm.at[tti_v.at[slot]], tbuf_v.at[slot],
                              gsems.at[slot]).wait()

    def compute(slot):
        def group_step(g, _):
            r0 = g * L
            for k in range(L):
                r = r0 + k
                for j in range(EMBED // L):
                    sl = pl.ds(j * L, L)
                    wbuf_v[slot, r, sl] = (wbuf_v[slot, r, sl]
                                           + pbuf_v[slot, r, sl]
                                           + tbuf_v[slot, r, sl])
            return _
        lax.fori_loop(0, CHUNK // L, group_step, 0, unroll=False)

    def store(c, slot):
        base = wbase + c * CHUNK
        pltpu.async_copy(wbuf_v.at[slot], out_hbm.at[pl.ds(base, CHUNK)],
                         ssems.at[slot])

    def wait_store(c, slot):
        base = wbase + c * CHUNK
        pltpu.make_async_copy(wbuf_v.at[slot], out_hbm.at[pl.ds(base, CHUNK)],
                              ssems.at[slot]).wait()

    issue(0, 0)
    issue(1, 1)

    def chunk_step(c, _):
        slot = c % NSLOT
        wait_gathers(c, slot)
        compute(slot)
        store(c, slot)
        wait_store(c, slot)
        issue(c + NSLOT, slot)
        return _

    lax.fori_loop(0, NCHUNK - NSLOT, chunk_step, 0, unroll=False)

    def tail_step(c, _):
        slot = c % NSLOT
        wait_gathers(c, slot)
        compute(slot)
        store(c, slot)
        wait_store(c, slot)
        return _

    lax.fori_loop(NCHUNK - NSLOT, NCHUNK, tail_step, 0, unroll=False)


def kernel(input_ids, token_type_ids, word_emb, pos_emb, tok_type_emb):
    ids = input_ids.reshape(N).astype(jnp.int32)
    tt = token_type_ids.reshape(N).astype(jnp.int32)

    mesh = plsc.VectorSubcoreMesh(core_axis_name="c", subcore_axis_name="s")
    out = pl.kernel(
        _body,
        mesh=mesh,
        out_type=jax.ShapeDtypeStruct((N, EMBED), jnp.float32),
        scratch_types=[
            pltpu.VMEM((NSLOT, CHUNK), jnp.int32),
            pltpu.VMEM((NSLOT, CHUNK), jnp.int32),
            pltpu.VMEM((SEQ,), jnp.int32),
            pltpu.VMEM((NSLOT, CHUNK, EMBED), jnp.float32),
            pltpu.VMEM((NSLOT, CHUNK, EMBED), jnp.float32),
            pltpu.VMEM((NSLOT, CHUNK, EMBED), jnp.float32),
            pltpu.SemaphoreType.DMA((NSLOT,)),
            pltpu.SemaphoreType.DMA((NSLOT,)),
        ],
    )(ids, tt, word_emb, pos_emb, tok_type_emb)
    return out.reshape(BATCH, SEQ, EMBED)


# speedup vs baseline: 12.8255x; 12.8255x over previous
import functools

import jax
import jax.numpy as jnp
from jax import lax
from jax.experimental import pallas as pl
from jax.experimental.pallas import tpu as pltpu
from jax.experimental.pallas import tpu_sc as plsc

VOCAB = 100000
EMBED = 128
BATCH = 1024
SEQ = 512
TYPE_VOCAB = 2

L = 16
NW = 32
N = BATCH * SEQ
ROWS_PER_W = N // NW
CHUNK = 128
NCHUNK = ROWS_PER_W // CHUNK
NSLOT = 2
POS_PERIOD = SEQ // CHUNK
GVECS = CHUNK // L


def _body(ids_hbm, tt_hbm, word_hbm, pos_hbm, ttab_hbm, out_hbm,
          idx_v, tti_v, ramp_v, ttab_v, wbuf_v, pbuf_v, gsems, ssems):
    wid = lax.axis_index("s") * 2 + lax.axis_index("c")
    wbase = wid * ROWS_PER_W

    pltpu.sync_copy(ttab_hbm, ttab_v)
    t0 = [ttab_v[0, pl.ds(j * L, L)] for j in range(EMBED // L)]
    dt = [ttab_v[1, pl.ds(j * L, L)] - t0[j] for j in range(EMBED // L)]

    for j in range(SEQ // L):
        ramp_v[pl.ds(j * L, L)] = lax.iota(jnp.int32, L) + (j * L)

    def issue(c, slot):
        base = wbase + c * CHUNK
        pos_off = (c % POS_PERIOD) * CHUNK
        pltpu.sync_copy(ids_hbm.at[pl.ds(base, CHUNK)], idx_v.at[slot])
        pltpu.sync_copy(tt_hbm.at[pl.ds(base, CHUNK)], tti_v.at[slot])
        pltpu.async_copy(word_hbm.at[idx_v.at[slot]], wbuf_v.at[slot],
                         gsems.at[slot])
        pltpu.async_copy(pos_hbm.at[ramp_v.at[pl.ds(pos_off, CHUNK)]],
                         pbuf_v.at[slot], gsems.at[slot])

    def wait_gathers(c, slot):
        base = wbase + c * CHUNK
        pos_off = (c % POS_PERIOD) * CHUNK
        pltpu.make_async_copy(word_hbm.at[idx_v.at[slot]], wbuf_v.at[slot],
                              gsems.at[slot]).wait()
        pltpu.make_async_copy(pos_hbm.at[ramp_v.at[pl.ds(pos_off, CHUNK)]],
                              pbuf_v.at[slot], gsems.at[slot]).wait()

    def compute(slot):
        def group_step(g, _):
            r0 = g * L
            fvec = tti_v[slot, pl.ds(r0, L)].astype(jnp.float32)
            for k in range(L):
                r = r0 + k
                f = fvec[k]
                for j in range(EMBED // L):
                    sl = pl.ds(j * L, L)
                    wbuf_v[slot, r, sl] = (wbuf_v[slot, r, sl]
                                           + pbuf_v[slot, r, sl]
                                           + (t0[j] + f * dt[j]))
            return _
        lax.fori_loop(0, CHUNK // L, group_step, 0, unroll=False)

    def store(c, slot):
        base = wbase + c * CHUNK
        pltpu.async_copy(wbuf_v.at[slot], out_hbm.at[pl.ds(base, CHUNK)],
                         ssems.at[slot])

    def wait_store(c, slot):
        base = wbase + c * CHUNK
        pltpu.make_async_copy(wbuf_v.at[slot], out_hbm.at[pl.ds(base, CHUNK)],
                              ssems.at[slot]).wait()

    issue(0, 0)
    issue(1, 1)

    def chunk_step(c, _):
        slot = c % NSLOT
        wait_gathers(c, slot)
        compute(slot)
        store(c, slot)
        wait_store(c, slot)
        issue(c + NSLOT, slot)
        return _

    lax.fori_loop(0, NCHUNK - NSLOT, chunk_step, 0, unroll=False)

    def tail_step(c, _):
        slot = c % NSLOT
        wait_gathers(c, slot)
        compute(slot)
        store(c, slot)
        wait_store(c, slot)
        return _

    lax.fori_loop(NCHUNK - NSLOT, NCHUNK, tail_step, 0, unroll=False)


def kernel(input_ids, token_type_ids, word_emb, pos_emb, tok_type_emb):
    ids = input_ids.reshape(N).astype(jnp.int32)
    tt = token_type_ids.reshape(N).astype(jnp.int32)

    mesh = plsc.VectorSubcoreMesh(core_axis_name="c", subcore_axis_name="s")
    out = pl.kernel(
        _body,
        mesh=mesh,
        out_type=jax.ShapeDtypeStruct((N, EMBED), jnp.float32),
        scratch_types=[
            pltpu.VMEM((NSLOT, CHUNK), jnp.int32),
            pltpu.VMEM((NSLOT, CHUNK), jnp.int32),
            pltpu.VMEM((SEQ,), jnp.int32),
            pltpu.VMEM((TYPE_VOCAB, EMBED), jnp.float32),
            pltpu.VMEM((NSLOT, CHUNK, EMBED), jnp.float32),
            pltpu.VMEM((NSLOT, CHUNK, EMBED), jnp.float32),
            pltpu.SemaphoreType.DMA((NSLOT,)),
            pltpu.SemaphoreType.DMA((NSLOT,)),
        ],
    )(ids, tt, word_emb, pos_emb, tok_type_emb)
    return out.reshape(BATCH, SEQ, EMBED)
